# Initial kernel scaffold; baseline (speedup 1.0000x reference)
import jax
import jax.numpy as jnp
from jax.experimental import pallas as pl


def kernel(x_ue, x_ap, edge_index_ue2ap, edge_attr_ue2ap, edge_index_ap2ue, edge_attr_ap2ue, params, power_params):
    raise NotImplementedError("write your pallas kernel here")


# speedup vs baseline: 1.0963x; 1.0963x over previous
import jax
import jax.numpy as jnp
from jax.experimental import pallas as pl

N_UE = 50000
N_AP = 50000
D = 32


def _head_body(x_ref, w1_ref, b1_ref, w2_ref, b2_ref, o_ref):
    x = x_ref[...]
    h = jax.nn.relu(
        jax.lax.dot_general(x, w1_ref[...], (((1,), (0,)), ((), ())),
                            preferred_element_type=jnp.float32) + b1_ref[...])
    p = jax.nn.sigmoid(
        jax.lax.dot_general(h, w2_ref[...], (((1,), (0,)), ((), ())),
                            preferred_element_type=jnp.float32) + b2_ref[...])
    o_ref[...] = jnp.concatenate([x[:, :1], p], axis=1)


def _head(x_ue, pm):
    n = x_ue.shape[0]
    blk = 2000
    grid = (n // blk,)
    return pl.pallas_call(
        _head_body,
        grid=grid,
        in_specs=[
            pl.BlockSpec((blk, D), lambda i: (i, 0)),
            pl.BlockSpec((D, 16), lambda i: (0, 0)),
            pl.BlockSpec((16,), lambda i: (0,)),
            pl.BlockSpec((16, 1), lambda i: (0, 0)),
            pl.BlockSpec((1,), lambda i: (0,)),
        ],
        out_specs=pl.BlockSpec((blk, 2), lambda i: (i, 0)),
        out_shape=jax.ShapeDtypeStruct((n, 2), jnp.float32),
    )(x_ue, pm['W1'], pm['b1'], pm['W2'], pm['b2'])


def _het_conv(x_ue, x_ap, ei_ua, ea_ua, ei_au, ea_au, p):
    src_u = x_ue[ei_ua[0]]
    m_ua = jax.nn.relu(jnp.concatenate([src_u, ea_ua], axis=1) @ p['W_ua'] + p['b_ua'])
    agg_ap = jax.ops.segment_sum(m_ua, ei_ua[1], num_segments=N_AP)
    ap_new = jax.nn.relu(agg_ap @ p['W_ap_upd'] + x_ap @ p['W_ap_self'] + p['b_ap'])
    src_a = x_ap[ei_au[0]]
    m_au = jax.nn.relu(jnp.concatenate([src_a, ea_au], axis=1) @ p['W_au'] + p['b_au'])
    agg_ue = jax.ops.segment_sum(m_au, ei_au[1], num_segments=N_UE)
    ue_new = jax.nn.relu(agg_ue @ p['W_ue_upd'] + x_ue @ p['W_ue_self'] + p['b_ue'])
    e_ua_new = jax.nn.relu(jnp.concatenate([src_u, x_ap[ei_ua[1]], ea_ua], axis=1) @ p['W_e_ua'] + p['b_e_ua'])
    e_au_new = jax.nn.relu(jnp.concatenate([x_ue[ei_au[1]], src_a, ea_au], axis=1) @ p['W_e_au'] + p['b_e_au'])
    return ue_new, ap_new, e_ua_new, e_au_new


def kernel(x_ue, x_ap, edge_index_ue2ap, edge_attr_ue2ap, edge_index_ap2ue, edge_attr_ap2ue, params, power_params):
    ea_ua = edge_attr_ue2ap
    ea_au = edge_attr_ap2ue
    for p in params:
        x_ue, x_ap, ea_ua, ea_au = _het_conv(x_ue, x_ap, edge_index_ue2ap, ea_ua, edge_index_ap2ue, ea_au, p)
    ue_out = _head(x_ue, power_params)
    return (ue_out, x_ap, ea_ua, ea_au)


# speedup vs baseline: 2.6355x; 2.4040x over previous
import functools

import jax
import jax.numpy as jnp
from jax import lax
from jax.experimental import pallas as pl
from jax.experimental.pallas import tpu as pltpu
from jax.experimental.pallas import tpu_sc as plsc

N_UE = 50000
N_AP = 50000
E = 1600000
D = 32

GW = 48
DW = 16

NS = 16
EPW = E // NS
SB = 100
C = 200
NSB = C // SB
NCH = EPW // C
ZB = 256
NZB = N_AP // ZB
ZREM = N_AP - NZB * ZB

_f32 = jnp.float32


_R = 2000


def _full(shape):
    return pl.BlockSpec(shape, lambda i: tuple(0 for _ in shape))


def _rows(w):
    return pl.BlockSpec((_R, w), lambda i: (i, 0))


def _pad_cols(x, w):
    return jnp.pad(x, ((0, 0), (0, w - x.shape[1])))


def _prep0_body(xu_ref, wu_ref, bua_ref, beua_ref, beau_ref,
                gue_ref, dap_ref, due_ref):
    xu = xu_ref[...]
    p = lax.dot_general(xu, wu_ref[...], (((1,), (0,)), ((), ())),
                        preferred_element_type=_f32)
    gue_ref[...] = _pad_cols(
        jnp.concatenate([p[:, 0:32] + bua_ref[...], p[:, 32:34]], axis=1), GW)
    due_ref[...] = _pad_cols(p[:, 34:36] + beau_ref[...], DW)
    dap_ref[...] = _pad_cols(
        jnp.broadcast_to(beua_ref[...], (xu.shape[0], 2)), DW)


def _prep0(x_ue, p0):
    wu = jnp.concatenate(
        [p0['W_ua'][:1], p0['W_e_ua'][:1], p0['W_e_au'][:1]], axis=1)
    return pl.pallas_call(
        _prep0_body,
        grid=(N_UE // _R,),
        in_specs=[_rows(1), _full((1, 36)), _full((1, D)),
                  _full((1, 2)), _full((1, 2))],
        out_specs=[_rows(GW), _rows(DW), _rows(DW)],
        out_shape=[jax.ShapeDtypeStruct((N_UE, GW), _f32),
                   jax.ShapeDtypeStruct((N_AP, DW), _f32),
                   jax.ShapeDtypeStruct((N_UE, DW), _f32)],
    )(x_ue, wu, p0['b_ua'][None], p0['b_e_ua'][None], p0['b_e_au'][None])


def _mid_body(has_xa, du,
              aggu_ref, agga_ref, xu_ref, xa_ref,
              wuu_ref, wus_ref, bue_ref, wau_ref, was_ref, bap_ref,
              wuall_ref, waall_ref, bua_ref, bau_ref, beua_ref, beau_ref,
              xun_ref, xan_ref, gue_ref, gap_ref, dap_ref, due_ref):
    dot = functools.partial(lax.dot_general,
                            dimension_numbers=(((1,), (0,)), ((), ())),
                            preferred_element_type=_f32)
    xu_new = dot(aggu_ref[...], wuu_ref[...]) + bue_ref[...]
    xu_new = xu_new + dot(xu_ref[...], wus_ref[...])
    xu_new = jax.nn.relu(xu_new)
    xa_new = dot(agga_ref[...], wau_ref[...]) + bap_ref[...]
    if has_xa:
        xa_new = xa_new + dot(xa_ref[...], was_ref[...])
    xa_new = jax.nn.relu(xa_new)
    xun_ref[...] = xu_new
    xan_ref[...] = xa_new
    pu = dot(xu_new, wuall_ref[...])
    pa = dot(xa_new, waall_ref[...])
    gue_ref[...] = _pad_cols(
        jnp.concatenate([pu[:, 0:32] + bua_ref[...], pu[:, 32:34]], axis=1), GW)
    gap_ref[...] = _pad_cols(
        jnp.concatenate([pa[:, 0:32] + bau_ref[...], pa[:, 32:34]], axis=1), GW)
    due_ref[...] = _pad_cols(pu[:, 34:36] + beau_ref[...], DW)
    dap_ref[...] = _pad_cols(pa[:, 34:36] + beua_ref[...], DW)


def _mid(agg_ue, agg_ap, xu_prev, xa_prev, pu, pp):
    has_xa = xa_prev is not None
    du = xu_prev.shape[1]
    wuall = jnp.concatenate(
        [pp['W_ua'][:D], pp['W_e_ua'][:D], pp['W_e_au'][:D]], axis=1)
    waall = jnp.concatenate(
        [pp['W_au'][:D], pp['W_e_au'][D:2 * D], pp['W_e_ua'][D:2 * D]], axis=1)
    xa_arg = xa_prev if has_xa else jnp.zeros((N_AP, 1), _f32)
    was_arg = pu['W_ap_self'] if has_xa else jnp.zeros((1, D), _f32)
    body = functools.partial(_mid_body, has_xa, du)
    return pl.pallas_call(
        body,
        grid=(N_UE // _R,),
        in_specs=[_rows(D), _rows(D), _rows(du), _rows(xa_arg.shape[1]),
                  _full((D, D)), _full((du, D)), _full((1, D)),
                  _full((D, D)), _full((was_arg.shape[0], D)), _full((1, D)),
                  _full((D, 36)), _full((D, 36)),
                  _full((1, D)), _full((1, D)), _full((1, 2)), _full((1, 2))],
        out_specs=[_rows(D), _rows(D), _rows(GW), _rows(GW),
                   _rows(DW), _rows(DW)],
        out_shape=[jax.ShapeDtypeStruct((N_UE, D), _f32),
                   jax.ShapeDtypeStruct((N_AP, D), _f32),
                   jax.ShapeDtypeStruct((N_UE, GW), _f32),
                   jax.ShapeDtypeStruct((N_AP, GW), _f32),
                   jax.ShapeDtypeStruct((N_AP, DW), _f32),
                   jax.ShapeDtypeStruct((N_UE, DW), _f32)],
    )(agg_ue, agg_ap, xu_prev, xa_arg,
      pu['W_ue_upd'], pu['W_ue_self'], pu['b_ue'][None],
      pu['W_ap_upd'], was_arg, pu['b_ap'][None],
      wuall, waall, pp['b_ua'][None], pp['b_au'][None],
      pp['b_e_ua'][None], pp['b_e_au'][None])


def _final_body(aggu_ref, agga_ref, xu_ref, xa_ref,
                wuu_ref, wus_ref, bue_ref, wau_ref, was_ref, bap_ref,
                w1_ref, b1_ref, w2_ref, b2_ref,
                out_ref, xan_ref):
    dot = functools.partial(lax.dot_general,
                            dimension_numbers=(((1,), (0,)), ((), ())),
                            preferred_element_type=_f32)
    xu_new = jax.nn.relu(dot(aggu_ref[...], wuu_ref[...])
                         + dot(xu_ref[...], wus_ref[...]) + bue_ref[...])
    xa_new = jax.nn.relu(dot(agga_ref[...], wau_ref[...])
                         + dot(xa_ref[...], was_ref[...]) + bap_ref[...])
    xan_ref[...] = xa_new
    h = jax.nn.relu(dot(xu_new, w1_ref[...]) + b1_ref[...])
    pw = jax.nn.sigmoid(dot(h, w2_ref[...]) + b2_ref[...])
    out_ref[...] = jnp.concatenate([xu_new[:, :1], pw], axis=1)


def _final(agg_ue, agg_ap, xu_prev, xa_prev, pu, pm):
    return pl.pallas_call(
        _final_body,
        grid=(N_UE // _R,),
        in_specs=[_rows(D), _rows(D), _rows(D), _rows(D),
                  _full((D, D)), _full((D, D)), _full((1, D)),
                  _full((D, D)), _full((D, D)), _full((1, D)),
                  _full((D, 16)), _full((1, 16)), _full((16, 1)),
                  _full((1, 1))],
        out_specs=[_rows(2), _rows(D)],
        out_shape=[jax.ShapeDtypeStruct((N_UE, 2), _f32),
                   jax.ShapeDtypeStruct((N_AP, D), _f32)],
    )(agg_ue, agg_ap, xu_prev, xa_prev,
      pu['W_ue_upd'], pu['W_ue_self'], pu['b_ue'][None],
      pu['W_ap_upd'], pu['W_ap_self'], pu['b_ap'][None],
      pm['W1'], pm['b1'][None], pm['W2'], pm['b2'][None])


def _pack_w(p, direction):
    if direction == 'ua':
        wmsg, wedge, b = p['W_ua'], p['W_e_ua'], p['b_ua']
    else:
        wmsg, wedge, b = p['W_au'], p['W_e_au'], p['b_au']
    wbot = wmsg[-2:]
    we = wedge[-2:]
    row4 = jnp.zeros((16,), _f32).at[0:4].set(
        jnp.stack([we[0, 0], we[1, 0], we[0, 1], we[1, 1]]))
    return jnp.concatenate([
        wbot[0].reshape(2, 16), wbot[1].reshape(2, 16),
        row4[None], b.reshape(2, 16), jnp.zeros((1, 16), _f32)], axis=0)


def _sc_edge_builder(has_src):
    mesh = plsc.VectorSubcoreMesh(core_axis_name="c", subcore_axis_name="s",
                                  num_cores=1)

    def body(*refs):
        if has_src:
            (gtab, dtab, shbm, dhbm, eahbm, wpk, zhbm,
             agg, enewhbm,
             sidx, didx, gbuf, dbuf, eabuf, mbuf, enew, wvm,
             shared) = refs
        else:
            (dtab, dhbm, eahbm, wpk, zhbm,
             agg, enewhbm,
             didx, dbuf, eabuf, mbuf, enew, wvm,
             shared) = refs
            gtab = shbm = sidx = gbuf = None
        sid = lax.axis_index("s")

        pltpu.sync_copy(wpk, wvm)

        def sweep(fn):
            for t in range(13):
                b = sid + NS * t

                @pl.when(b < NZB)
                def _():
                    fn(b * ZB, ZB)

            @pl.when(sid == 0)
            def _():
                fn(NZB * ZB, ZREM)

        sweep(lambda r0, n: pltpu.sync_copy(
            zhbm.at[pl.ds(0, n)], shared.at[pl.ds(r0, n)]))
        plsc.subcore_barrier()

        iota16 = lax.iota(jnp.int32, 16)
        zeros16i = jnp.zeros((16,), jnp.int32)
        ones16i = jnp.full((16,), 1, jnp.int32)

        if True:
            w0a = wvm[0]
            w0b = wvm[1]
            w1a = wvm[2]
            w1b = wvm[3]
            wr4 = wvm[4]
            we00 = wr4[0]
            we10 = wr4[1]
            we01 = wr4[2]
            we11 = wr4[3]
            if gtab is None:
                bv0 = wvm[5]
                bv1 = wvm[6]
            base_row = sid * (EPW // SB)

            def chunk(c, carry):
                r0 = base_row + c * NSB
                e0 = r0 * SB
                if gtab is not None:
                    pltpu.sync_copy(shbm.at[pl.ds(r0, NSB)], sidx)
                pltpu.sync_copy(dhbm.at[pl.ds(r0, NSB)], didx)
                pltpu.sync_copy(eahbm.at[pl.ds(2 * e0, 2 * C)], eabuf)
                for j in range(NSB):
                    if gtab is not None:
                        pltpu.sync_copy(gtab.at[sidx.at[j]],
                                        gbuf.at[pl.ds(j * SB, SB)])
                    pltpu.sync_copy(dtab.at[didx.at[j]],
                                    dbuf.at[pl.ds(j * SB, SB)])

                @plsc.parallel_loop(0, C // 8, 1, unroll=2)
                def _(g):
                    av = eabuf[pl.ds(g * 16, 16)]
                    for k in range(8):
                        e = g * 8 + k
                        a0 = av[2 * k]
                        a1 = av[2 * k + 1]
                        if gtab is not None:
                            m0 = jnp.maximum(
                                gbuf[e, pl.ds(0, 16)] + a0 * w0a + a1 * w1a,
                                0.0)
                            m1 = jnp.maximum(
                                gbuf[e, pl.ds(16, 16)] + a0 * w0b + a1 * w1b,
                                0.0)
                        else:
                            m0 = jnp.maximum(bv0 + a0 * w0a + a1 * w1a, 0.0)
                            m1 = jnp.maximum(bv1 + a0 * w0b + a1 * w1b, 0.0)
                        mbuf[e, pl.ds(0, 16)] = m0
                        mbuf[e, pl.ds(16, 16)] = m1

                @plsc.parallel_loop(0, (C + 15) // 16, 1, unroll=4)
                def _(g):
                    rows = g * 16 + iota16
                    msk = rows < C
                    rows = jnp.minimum(rows, C - 1)
                    rows2 = rows * 2
                    a0v = plsc.load_gather(eabuf, [rows2])
                    a1v = plsc.load_gather(eabuf, [rows2 + 1])
                    d0v = plsc.load_gather(dbuf, [rows, zeros16i])
                    d1v = plsc.load_gather(dbuf, [rows, ones16i])
                    o0 = d0v + a0v * we00 + a1v * we10
                    o1 = d1v + a0v * we01 + a1v * we11
                    if gtab is not None:
                        o0 = o0 + plsc.load_gather(
                            gbuf, [rows, jnp.full((16,), 32, jnp.int32)])
                        o1 = o1 + plsc.load_gather(
                            gbuf, [rows, jnp.full((16,), 33, jnp.int32)])
                    plsc.store_scatter(enew, [rows2],
                                       jnp.maximum(o0, 0.0), mask=msk)
                    plsc.store_scatter(enew, [rows2 + 1],
                                       jnp.maximum(o1, 0.0), mask=msk)

                for j in range(NSB):
                    pltpu.sync_copy(mbuf.at[pl.ds(j * SB, SB)],
                                    shared.at[didx.at[j]], add=True)
                pltpu.sync_copy(enew, enewhbm.at[pl.ds(2 * e0, 2 * C)])
                return carry

            lax.fori_loop(0, NCH, chunk, 0)

        plsc.subcore_barrier()
        sweep(lambda r0, n: pltpu.sync_copy(
            shared.at[pl.ds(r0, n)], agg.at[pl.ds(r0, n)]))

    scratch = [
        pltpu.VMEM((NSB, SB), jnp.int32),
        pltpu.VMEM((NSB, SB), jnp.int32),
        pltpu.VMEM((C, GW), _f32),
        pltpu.VMEM((C, DW), _f32),
        pltpu.VMEM((2 * C,), _f32),
        pltpu.VMEM((C, D), _f32),
        pltpu.VMEM((2 * C,), _f32),
        pltpu.VMEM((8, 16), _f32),
        pltpu.VMEM_SHARED((N_AP, D), _f32),
    ]
    if not has_src:
        del scratch[2]
        del scratch[0]
    return pl.kernel(
        body,
        out_type=[jax.ShapeDtypeStruct((N_AP, D), _f32),
                  jax.ShapeDtypeStruct((2 * E,), _f32)],
        mesh=mesh,
        scratch_types=scratch,
        compiler_params=pltpu.CompilerParams(needs_layout_passes=False,
                                             use_tc_tiling_on_sc=False),
        name="sc_edge" + ("_full" if has_src else "_nosrc"),
    )


_sc_edge_full = _sc_edge_builder(True)
_sc_edge_nosrc = _sc_edge_builder(False)


def kernel(x_ue, x_ap, edge_index_ue2ap, edge_attr_ue2ap,
           edge_index_ap2ue, edge_attr_ap2ue, params, power_params):
    ei_ua = edge_index_ue2ap.astype(jnp.int32)
    ei_au = edge_index_ap2ue.astype(jnp.int32)
    sua = ei_ua[0].reshape(E // SB, SB)
    dua = ei_ua[1].reshape(E // SB, SB)
    sau = ei_au[0].reshape(E // SB, SB)
    dau = ei_au[1].reshape(E // SB, SB)
    ea_ua = edge_attr_ue2ap.reshape(-1)
    ea_au = edge_attr_ap2ue.reshape(-1)

    p0 = params[0]
    zblk = jnp.zeros((ZB, D), _f32)
    gue, dap, due = _prep0(x_ue, p0)
    agg_ap, ea_ua = _sc_edge_full(gue, dap, sua, dua, ea_ua,
                                  _pack_w(p0, 'ua'), zblk)
    agg_ue, ea_au = _sc_edge_nosrc(due, dau, ea_au, _pack_w(p0, 'au'), zblk)

    xu, xa = x_ue, None
    for i in (1, 2, 3):
        pu, pp = params[i - 1], params[i]
        xu, xa, gue, gap, dap, due = _mid(agg_ue, agg_ap, xu, xa, pu, pp)
        agg_ap, ea_ua = _sc_edge_full(gue, dap, sua, dua, ea_ua,
                                      _pack_w(pp, 'ua'), zblk)
        agg_ue, ea_au = _sc_edge_full(gap, due, sau, dau, ea_au,
                                      _pack_w(pp, 'au'), zblk)

    ue_out, xa4 = _final(agg_ue, agg_ap, xu, xa, params[3], power_params)
    return (ue_out, xa4, ea_ua.reshape(E, 2), ea_au.reshape(E, 2))


# speedup vs baseline: 6.4099x; 2.4321x over previous
import functools

import jax
import jax.numpy as jnp
from jax import lax
from jax.experimental import pallas as pl
from jax.experimental.pallas import tpu as pltpu
from jax.experimental.pallas import tpu_sc as plsc

N_UE = 50000
N_AP = 50000
E = 1600000
D = 32

GW = 40
DW = 8

NS = 16
EPW = E // NS
SB = 100
C = 200
NSB = C // SB
NCH = EPW // C
ZB = 256
NZB = N_AP // ZB
ZREM = N_AP - NZB * ZB

_f32 = jnp.float32


_R = 2000


def _b16(x):
    return x.astype(jnp.bfloat16).astype(_f32)


def _full(shape):
    return pl.BlockSpec(shape, lambda i: tuple(0 for _ in shape))


def _rows(w):
    return pl.BlockSpec((_R, w), lambda i: (i, 0))


def _pad_cols(x, w):
    return jnp.pad(x, ((0, 0), (0, w - x.shape[1])))


def _prep0_body(xu_ref, wu_ref, bua_ref, beua_ref, beau_ref,
                gue_ref, dap_ref, due_ref):
    xu = xu_ref[...]
    p = lax.dot_general(_b16(xu), _b16(wu_ref[...]),
                        (((1,), (0,)), ((), ())),
                        precision=lax.Precision.HIGHEST,
                        preferred_element_type=_f32)
    gue_ref[...] = _pad_cols(
        jnp.concatenate([p[:, 0:32] + bua_ref[...], p[:, 32:34]], axis=1), GW)
    due_ref[...] = _pad_cols(p[:, 34:36] + beau_ref[...], DW)
    dap_ref[...] = _pad_cols(
        jnp.broadcast_to(beua_ref[...], (xu.shape[0], 2)), DW)


def _prep0(x_ue, p0):
    wu = jnp.concatenate(
        [p0['W_ua'][:1], p0['W_e_ua'][:1], p0['W_e_au'][:1]], axis=1)
    return pl.pallas_call(
        _prep0_body,
        grid=(N_UE // _R,),
        in_specs=[_rows(1), _full((1, 36)), _full((1, D)),
                  _full((1, 2)), _full((1, 2))],
        out_specs=[_rows(GW), _rows(DW), _rows(DW)],
        out_shape=[jax.ShapeDtypeStruct((N_UE, GW), _f32),
                   jax.ShapeDtypeStruct((N_AP, DW), _f32),
                   jax.ShapeDtypeStruct((N_UE, DW), _f32)],
    )(x_ue, wu, p0['b_ua'][None], p0['b_e_ua'][None], p0['b_e_au'][None])


def _mid_body(has_xa, du,
              aggu_ref, agga_ref, xu_ref, xa_ref,
              wuu_ref, wus_ref, bue_ref, wau_ref, was_ref, bap_ref,
              wuall_ref, waall_ref, bua_ref, bau_ref, beua_ref, beau_ref,
              xun_ref, xan_ref, gue_ref, gap_ref, dap_ref, due_ref):
    def dot(a, b):
        return lax.dot_general(_b16(a), _b16(b), (((1,), (0,)), ((), ())),
                               precision=lax.Precision.HIGHEST,
                               preferred_element_type=_f32)
    xu_new = dot(aggu_ref[...], wuu_ref[...]) + bue_ref[...]
    xu_new = xu_new + dot(xu_ref[...], wus_ref[...])
    xu_new = jax.nn.relu(xu_new)
    xa_new = dot(agga_ref[...], wau_ref[...]) + bap_ref[...]
    if has_xa:
        xa_new = xa_new + dot(xa_ref[...], was_ref[...])
    xa_new = jax.nn.relu(xa_new)
    xun_ref[...] = xu_new
    xan_ref[...] = xa_new
    pu = dot(xu_new, wuall_ref[...])
    pa = dot(xa_new, waall_ref[...])
    gue_ref[...] = _pad_cols(
        jnp.concatenate([pu[:, 0:32] + bua_ref[...], pu[:, 32:34]], axis=1), GW)
    gap_ref[...] = _pad_cols(
        jnp.concatenate([pa[:, 0:32] + bau_ref[...], pa[:, 32:34]], axis=1), GW)
    due_ref[...] = _pad_cols(pu[:, 34:36] + beau_ref[...], DW)
    dap_ref[...] = _pad_cols(pa[:, 34:36] + beua_ref[...], DW)


def _mid(agg_ue, agg_ap, xu_prev, xa_prev, pu, pp):
    has_xa = xa_prev is not None
    du = xu_prev.shape[1]
    wuall = jnp.concatenate(
        [pp['W_ua'][:D], pp['W_e_ua'][:D], pp['W_e_au'][:D]], axis=1)
    waall = jnp.concatenate(
        [pp['W_au'][:D], pp['W_e_au'][D:2 * D], pp['W_e_ua'][D:2 * D]], axis=1)
    xa_arg = xa_prev if has_xa else jnp.zeros((N_AP, 1), _f32)
    was_arg = pu['W_ap_self'] if has_xa else jnp.zeros((1, D), _f32)
    body = functools.partial(_mid_body, has_xa, du)
    return pl.pallas_call(
        body,
        grid=(N_UE // _R,),
        in_specs=[_rows(D), _rows(D), _rows(du), _rows(xa_arg.shape[1]),
                  _full((D, D)), _full((du, D)), _full((1, D)),
                  _full((D, D)), _full((was_arg.shape[0], D)), _full((1, D)),
                  _full((D, 36)), _full((D, 36)),
                  _full((1, D)), _full((1, D)), _full((1, 2)), _full((1, 2))],
        out_specs=[_rows(D), _rows(D), _rows(GW), _rows(GW),
                   _rows(DW), _rows(DW)],
        out_shape=[jax.ShapeDtypeStruct((N_UE, D), _f32),
                   jax.ShapeDtypeStruct((N_AP, D), _f32),
                   jax.ShapeDtypeStruct((N_UE, GW), _f32),
                   jax.ShapeDtypeStruct((N_AP, GW), _f32),
                   jax.ShapeDtypeStruct((N_AP, DW), _f32),
                   jax.ShapeDtypeStruct((N_UE, DW), _f32)],
    )(agg_ue, agg_ap, xu_prev, xa_arg,
      pu['W_ue_upd'], pu['W_ue_self'], pu['b_ue'][None],
      pu['W_ap_upd'], was_arg, pu['b_ap'][None],
      wuall, waall, pp['b_ua'][None], pp['b_au'][None],
      pp['b_e_ua'][None], pp['b_e_au'][None])


def _final_body(aggu_ref, agga_ref, xu_ref, xa_ref,
                wuu_ref, wus_ref, bue_ref, wau_ref, was_ref, bap_ref,
                w1_ref, b1_ref, w2_ref, b2_ref,
                out_ref, xan_ref):
    def dot(a, b):
        return lax.dot_general(_b16(a), _b16(b), (((1,), (0,)), ((), ())),
                               precision=lax.Precision.HIGHEST,
                               preferred_element_type=_f32)
    xu_new = jax.nn.relu(dot(aggu_ref[...], wuu_ref[...])
                         + dot(xu_ref[...], wus_ref[...]) + bue_ref[...])
    xa_new = jax.nn.relu(dot(agga_ref[...], wau_ref[...])
                         + dot(xa_ref[...], was_ref[...]) + bap_ref[...])
    xan_ref[...] = xa_new
    h = jax.nn.relu(dot(xu_new, w1_ref[...]) + b1_ref[...])
    pw = jax.nn.sigmoid(dot(h, w2_ref[...]) + b2_ref[...])
    out_ref[...] = jnp.concatenate([xu_new[:, :1], pw], axis=1)


def _final(agg_ue, agg_ap, xu_prev, xa_prev, pu, pm):
    return pl.pallas_call(
        _final_body,
        grid=(N_UE // _R,),
        in_specs=[_rows(D), _rows(D), _rows(D), _rows(D),
                  _full((D, D)), _full((D, D)), _full((1, D)),
                  _full((D, D)), _full((D, D)), _full((1, D)),
                  _full((D, 16)), _full((1, 16)), _full((16, 1)),
                  _full((1, 1))],
        out_specs=[_rows(2), _rows(D)],
        out_shape=[jax.ShapeDtypeStruct((N_UE, 2), _f32),
                   jax.ShapeDtypeStruct((N_AP, D), _f32)],
    )(agg_ue, agg_ap, xu_prev, xa_prev,
      pu['W_ue_upd'], pu['W_ue_self'], pu['b_ue'][None],
      pu['W_ap_upd'], pu['W_ap_self'], pu['b_ap'][None],
      pm['W1'], pm['b1'][None], pm['W2'], pm['b2'][None])


def _pack_w(p, direction):
    if direction == 'ua':
        wmsg, wedge, b = p['W_ua'], p['W_e_ua'], p['b_ua']
    else:
        wmsg, wedge, b = p['W_au'], p['W_e_au'], p['b_au']
    wbot = wmsg[-2:]
    we = wedge[-2:]
    row4 = jnp.zeros((16,), _f32).at[0:4].set(
        jnp.stack([we[0, 0], we[1, 0], we[0, 1], we[1, 1]]))
    return jnp.concatenate([
        wbot[0].reshape(2, 16), wbot[1].reshape(2, 16),
        row4[None], b.reshape(2, 16), jnp.zeros((1, 16), _f32)], axis=0)


def _sc_edge_builder(has_src):
    mesh = plsc.VectorSubcoreMesh(core_axis_name="c", subcore_axis_name="s",
                                  num_cores=1)

    def body(*refs):
        if has_src:
            (gtab, dtab, shbm, dhbm, eahbm, wpk, zhbm,
             agg, enewhbm,
             sidx, didx, gbuf, dbuf, eabuf, mbuf, enew, wvm,
             si0, si1, si2, si3, sg0, sg1, ss, so0, so1,
             shared) = refs
        else:
            (dtab, dhbm, eahbm, wpk, zhbm,
             agg, enewhbm,
             sidx, didx, gbuf, dbuf, eabuf, mbuf, enew, wvm,
             si0, si1, si2, si3, sg0, sg1, ss, so0, so1,
             shared) = refs
            gtab = shbm = None
        sem_i = (si0, si1, si2, si3)
        sem_g = (sg0, sg1)
        sem_o = (so0, so1)
        sid = lax.axis_index("s")

        pltpu.sync_copy(wpk, wvm)

        def sweep(fn):
            for t in range(13):
                b = sid + NS * t

                @pl.when(b < NZB)
                def _():
                    fn(b * ZB, ZB)

            @pl.when(sid == 0)
            def _():
                fn(NZB * ZB, ZREM)

        sweep(lambda r0, n: pltpu.sync_copy(
            zhbm.at[pl.ds(0, n)], shared.at[pl.ds(r0, n)]))
        plsc.subcore_barrier()

        iota16 = lax.iota(jnp.int32, 16)
        zeros16i = jnp.zeros((16,), jnp.int32)
        ones16i = jnp.full((16,), 1, jnp.int32)

        def bf16r(v):
            u = plsc.bitcast(v, jnp.uint32)
            u = (u + jnp.uint32(0x7FFF) + ((u >> jnp.uint32(16))
                                           & jnp.uint32(1)))
            u = u & jnp.uint32(0xFFFF0000)
            return plsc.bitcast(u, _f32)

        w0a = bf16r(wvm[0])
        w0b = bf16r(wvm[1])
        w1a = bf16r(wvm[2])
        w1b = bf16r(wvm[3])
        wr4 = bf16r(wvm[4])
        we00 = wr4[0]
        we10 = wr4[1]
        we01 = wr4[2]
        we11 = wr4[3]
        if gtab is None:
            bv0 = wvm[5]
            bv1 = wvm[6]
        base_row = sid * (EPW // SB)
        nrows = E // SB

        def idx_dma(c, p, start):
            r0 = jnp.minimum(base_row + c * NSB, nrows - NSB)
            e0 = r0 * SB
            dd = [pltpu.make_async_copy(dhbm.at[pl.ds(r0, NSB)],
                                        didx.at[p], sem_i[p]),
                  pltpu.make_async_copy(eahbm.at[pl.ds(2 * e0, 2 * C)],
                                        eabuf.at[p], sem_i[p])]
            if has_src:
                dd.append(pltpu.make_async_copy(shbm.at[pl.ds(r0, NSB)],
                                                sidx.at[p], sem_i[p]))
            for d_ in dd:
                d_.start() if start else d_.wait()

        def gather_dma(p4, p2, start):
            for j in range(NSB):
                dd = [pltpu.make_async_copy(dtab.at[didx.at[p4, j]],
                                            dbuf.at[p2, pl.ds(j * SB, SB)],
                                            sem_g[p2])]
                if has_src:
                    dd.append(pltpu.make_async_copy(
                        gtab.at[sidx.at[p4, j]],
                        gbuf.at[p2, pl.ds(j * SB, SB)], sem_g[p2]))
                for d_ in dd:
                    d_.start() if start else d_.wait()

        def scat_dma(p4, start):
            for j in range(NSB):
                d_ = pltpu.make_async_copy(mbuf.at[pl.ds(j * SB, SB)],
                                           shared.at[didx.at[p4, j]], ss)
                d_.start(add=True) if start else d_.wait()

        def out_dma(c, p2, start):
            e0 = jnp.minimum(base_row + c * NSB, nrows - NSB) * SB
            d_ = pltpu.make_async_copy(enew.at[p2],
                                       enewhbm.at[pl.ds(2 * e0, 2 * C)],
                                       sem_o[p2])
            d_.start() if start else d_.wait()

        def compute(p4, p2):
            @plsc.parallel_loop(0, C // 8, 1, unroll=2)
            def _(g):
                av = bf16r(eabuf[p4, pl.ds(g * 16, 16)])
                for k in range(8):
                    e = g * 8 + k
                    a0 = av[2 * k]
                    a1 = av[2 * k + 1]
                    if gtab is not None:
                        m0 = jnp.maximum(
                            gbuf[p2, e, pl.ds(0, 16)] + a0 * w0a + a1 * w1a,
                            0.0)
                        m1 = jnp.maximum(
                            gbuf[p2, e, pl.ds(16, 16)] + a0 * w0b + a1 * w1b,
                            0.0)
                    else:
                        m0 = jnp.maximum(bv0 + a0 * w0a + a1 * w1a, 0.0)
                        m1 = jnp.maximum(bv1 + a0 * w0b + a1 * w1b, 0.0)
                    mbuf[e, pl.ds(0, 16)] = m0
                    mbuf[e, pl.ds(16, 16)] = m1

            @plsc.parallel_loop(0, (C + 15) // 16, 1, unroll=4)
            def _(g):
                rows = g * 16 + iota16
                msk = rows < C
                rows = jnp.minimum(rows, C - 1)
                rows2 = rows * 2
                a0v = bf16r(plsc.load_gather(eabuf.at[p4], [rows2]))
                a1v = bf16r(plsc.load_gather(eabuf.at[p4], [rows2 + 1]))
                d0v = plsc.load_gather(dbuf.at[p2], [rows, zeros16i])
                d1v = plsc.load_gather(dbuf.at[p2], [rows, ones16i])
                o0 = d0v + a0v * we00 + a1v * we10
                o1 = d1v + a0v * we01 + a1v * we11
                if gtab is not None:
                    o0 = o0 + plsc.load_gather(
                        gbuf.at[p2], [rows, jnp.full((16,), 32, jnp.int32)])
                    o1 = o1 + plsc.load_gather(
                        gbuf.at[p2], [rows, jnp.full((16,), 33, jnp.int32)])
                plsc.store_scatter(enew.at[p2], [rows2],
                                   jnp.maximum(o0, 0.0), mask=msk)
                plsc.store_scatter(enew.at[p2], [rows2 + 1],
                                   jnp.maximum(o1, 0.0), mask=msk)

        idx_dma(0, 0, True)
        idx_dma(1, 1, True)
        idx_dma(0, 0, False)
        gather_dma(0, 0, True)

        def outer(ii, carry):
            for k in range(4):
                i = ii * 4 + k
                p4 = k
                p2 = k % 2
                idx_dma(0, (k + 1) % 4, False)
                gather_dma((k + 1) % 4, (k + 1) % 2, True)
                gather_dma(p4, p2, False)
                if k == 0:
                    @pl.when(ii > 0)
                    def _():
                        scat_dma(3, False)
                else:
                    scat_dma(k - 1, False)
                if k < 2:
                    @pl.when(ii > 0)
                    def _():
                        out_dma(0, p2, False)
                else:
                    out_dma(0, p2, False)
                compute(p4, p2)
                idx_dma(i + 2, (k + 2) % 4, True)
                scat_dma(p4, True)
                out_dma(i, p2, True)
            return carry

        lax.fori_loop(0, NCH // 4, outer, 0)

        scat_dma(3, False)
        out_dma(0, 0, False)
        out_dma(0, 1, False)
        gather_dma(0, 0, False)
        idx_dma(0, 1, False)

        plsc.subcore_barrier()
        sweep(lambda r0, n: pltpu.sync_copy(
            shared.at[pl.ds(r0, n)], agg.at[pl.ds(r0, n)]))

    scratch = [
        pltpu.VMEM((4, NSB, SB), jnp.int32),
        pltpu.VMEM((4, NSB, SB), jnp.int32),
        pltpu.VMEM((2, C, GW), _f32),
        pltpu.VMEM((2, C, DW), _f32),
        pltpu.VMEM((4, 2 * C), _f32),
        pltpu.VMEM((C, D), _f32),
        pltpu.VMEM((2, 2 * C), _f32),
        pltpu.VMEM((8, 16), _f32),
        pltpu.SemaphoreType.DMA,
        pltpu.SemaphoreType.DMA,
        pltpu.SemaphoreType.DMA,
        pltpu.SemaphoreType.DMA,
        pltpu.SemaphoreType.DMA,
        pltpu.SemaphoreType.DMA,
        pltpu.SemaphoreType.DMA,
        pltpu.SemaphoreType.DMA,
        pltpu.SemaphoreType.DMA,
        pltpu.VMEM_SHARED((N_AP, D), _f32),
    ]
    return pl.kernel(
        body,
        out_type=[jax.ShapeDtypeStruct((N_AP, D), _f32),
                  jax.ShapeDtypeStruct((2 * E,), _f32)],
        mesh=mesh,
        scratch_types=scratch,
        compiler_params=pltpu.CompilerParams(needs_layout_passes=False,
                                             use_tc_tiling_on_sc=False),
        name="sc_edge" + ("_full" if has_src else "_nosrc"),
    )


_sc_edge_full = _sc_edge_builder(True)
_sc_edge_nosrc = _sc_edge_builder(False)


def kernel(x_ue, x_ap, edge_index_ue2ap, edge_attr_ue2ap,
           edge_index_ap2ue, edge_attr_ap2ue, params, power_params):
    ei_ua = edge_index_ue2ap.astype(jnp.int32)
    ei_au = edge_index_ap2ue.astype(jnp.int32)
    sua = ei_ua[0].reshape(E // SB, SB)
    dua = ei_ua[1].reshape(E // SB, SB)
    sau = ei_au[0].reshape(E // SB, SB)
    dau = ei_au[1].reshape(E // SB, SB)
    ea_ua = edge_attr_ue2ap.reshape(-1)
    ea_au = edge_attr_ap2ue.reshape(-1)

    p0 = params[0]
    zblk = jnp.zeros((ZB, D), _f32)
    gue, dap, due = _prep0(x_ue, p0)
    agg_ap, ea_ua = _sc_edge_full(gue, dap, sua, dua, ea_ua,
                                  _pack_w(p0, 'ua'), zblk)
    agg_ue, ea_au = _sc_edge_nosrc(due, dau, ea_au, _pack_w(p0, 'au'), zblk)

    xu, xa = x_ue, None
    for i in (1, 2, 3):
        pu, pp = params[i - 1], params[i]
        xu, xa, gue, gap, dap, due = _mid(agg_ue, agg_ap, xu, xa, pu, pp)
        agg_ap, ea_ua = _sc_edge_full(gue, dap, sua, dua, ea_ua,
                                      _pack_w(pp, 'ua'), zblk)
        agg_ue, ea_au = _sc_edge_full(gap, due, sau, dau, ea_au,
                                      _pack_w(pp, 'au'), zblk)

    ue_out, xa4 = _final(agg_ue, agg_ap, xu, xa, params[3], power_params)
    return (ue_out, xa4, ea_ua.reshape(E, 2), ea_au.reshape(E, 2))
